# baseline (speedup 1.0000x reference)
import functools

import jax
import jax.numpy as jnp
from jax import lax
from jax.experimental import pallas as pl
from jax.experimental.pallas import tpu as pltpu
from jax.experimental.pallas import tpu_sc as plsc

N = 10000
NP = 10240
E = 320000
D = 128
M = 64

NC = 2
NS = 16
NW = NC * NS
C = 128
EP = 327680
RD = EP // NW // C
NB = EP // NW // C
CH = 16
NCH = NB // CH
RPT = NP // NS
TRASH = NP - 1
BLK = 2048
GRID = NP // BLK

_sc_mesh = plsc.VectorSubcoreMesh(
    core_axis_name="c", subcore_axis_name="s", num_cores=NC, num_subcores=NS)
_sc_params = pltpu.CompilerParams(needs_layout_passes=False)

_zero16 = functools.partial(jnp.zeros, (16,), jnp.float32)


@functools.partial(
    pl.kernel,
    out_type=jax.ShapeDtypeStruct((NC * NP,), jnp.float32),
    mesh=_sc_mesh,
    scratch_types=[
        pltpu.VMEM((RD, C), jnp.int32),
        pltpu.VMEM((NP,), jnp.float32),
        pltpu.VMEM((RPT,), jnp.float32),
        pltpu.VMEM((RPT,), jnp.float32),
        pltpu.VMEM_SHARED((NS * NP,), jnp.float32),
    ],
    compiler_params=_sc_params,
)
def _deg_kernel(dst_hbm, deg_out, idx_v, hist_v, tmp_v, acc_v, shared_s):
    c = lax.axis_index("c")
    s = lax.axis_index("s")
    pltpu.sync_copy(dst_hbm.at[c * NS + s], idx_v)

    def _zero_hist(i, _):
        hist_v[pl.ds(i * 16, 16)] = _zero16()
        return 0
    lax.fori_loop(0, NP // 16, _zero_hist, 0)

    ones = jnp.ones((16,), jnp.float32)

    def _count(r, _):
        for k in range(C // 16):
            plsc.addupdate_scatter(hist_v, [idx_v[r, pl.ds(k * 16, 16)]], ones)
        return 0
    lax.fori_loop(0, RD, _count, 0)

    pltpu.sync_copy(hist_v, shared_s.at[pl.ds(s * NP, NP)])
    plsc.subcore_barrier()

    base = s * RPT

    def _zero_acc(i, _):
        acc_v[pl.ds(i * 16, 16)] = _zero16()
        return 0
    lax.fori_loop(0, RPT // 16, _zero_acc, 0)

    for k in range(NS):
        pltpu.sync_copy(shared_s.at[pl.ds(k * NP + base, RPT)], tmp_v)

        def _addv(i, _):
            acc_v[pl.ds(i * 16, 16)] = (
                acc_v[pl.ds(i * 16, 16)] + tmp_v[pl.ds(i * 16, 16)])
            return 0
        lax.fori_loop(0, RPT // 16, _addv, 0)

    pltpu.sync_copy(acc_v, deg_out.at[pl.ds(c * NP + base, RPT)])


@functools.partial(
    pl.kernel,
    out_type=jax.ShapeDtypeStruct((NC, NP, D), jnp.float32),
    mesh=_sc_mesh,
    scratch_types=[
        pltpu.VMEM((2, CH, C), jnp.int32),
        pltpu.VMEM((2, CH, C), jnp.int32),
        pltpu.VMEM((C, D), jnp.float32),
        pltpu.VMEM((C, D), jnp.float32),
        pltpu.VMEM_SHARED((NP, D), jnp.float32),
        pltpu.SemaphoreType.DMA,
        pltpu.SemaphoreType.DMA,
        pltpu.SemaphoreType.DMA,
        pltpu.SemaphoreType.DMA,
    ],
    compiler_params=_sc_params,
)
def _prop_kernel(y_hbm, srcr, dstr, accp, sidx_v, didx_v, rows_a, rows_b,
                 acc_s, gsem_a, gsem_b, isem_s, isem_d):
    c = lax.axis_index("c")
    s = lax.axis_index("s")
    wid = c * NS + s

    def _zrow(i, _):
        for k in range(D // 16):
            rows_a[i, pl.ds(k * 16, 16)] = _zero16()
        return 0
    lax.fori_loop(0, C, _zrow, 0)

    base = s * RPT
    for t in range(RPT // C):
        pltpu.sync_copy(rows_a, acc_s.at[pl.ds(base + t * C, C)])

    pltpu.sync_copy(srcr.at[wid, pl.ds(0, CH)], sidx_v.at[0])
    pltpu.sync_copy(dstr.at[wid, pl.ds(0, CH)], didx_v.at[0])
    pltpu.async_copy(y_hbm.at[sidx_v.at[0, 0]], rows_a, gsem_a)
    plsc.subcore_barrier()

    def _wait_rows(buf, sem):
        pltpu.make_async_copy(y_hbm.at[pl.ds(0, C)], buf, sem).wait()

    for ci in range(NCH):
        p = ci % 2
        nxt = ci + 1
        if nxt < NCH:
            pltpu.async_copy(srcr.at[wid, pl.ds(nxt * CH, CH)],
                             sidx_v.at[nxt % 2], isem_s)
            pltpu.async_copy(dstr.at[wid, pl.ds(nxt * CH, CH)],
                             didx_v.at[nxt % 2], isem_d)

        def _pair(j, _):
            ra = 2 * j
            rb = 2 * j + 1
            _wait_rows(rows_a, gsem_a)
            pltpu.async_copy(y_hbm.at[sidx_v.at[p, rb]], rows_b, gsem_b)
            pltpu.sync_copy(rows_a, acc_s.at[didx_v.at[p, ra]], add=True)
            _wait_rows(rows_b, gsem_b)

            @pl.when(j < CH // 2 - 1)
            def _():
                pltpu.async_copy(y_hbm.at[sidx_v.at[p, 2 * j + 2]], rows_a,
                                 gsem_a)
            pltpu.sync_copy(rows_b, acc_s.at[didx_v.at[p, rb]], add=True)
            return 0
        lax.fori_loop(0, CH // 2, _pair, 0)

        if nxt < NCH:
            q = nxt % 2
            pltpu.make_async_copy(srcr.at[wid, pl.ds(nxt * CH, CH)],
                                  sidx_v.at[q], isem_s).wait()
            pltpu.make_async_copy(dstr.at[wid, pl.ds(nxt * CH, CH)],
                                  didx_v.at[q], isem_d).wait()
            pltpu.async_copy(y_hbm.at[sidx_v.at[q, 0]], rows_a, gsem_a)

    plsc.subcore_barrier()
    for t in range(RPT // C):
        sl = pl.ds(base + t * C, C)
        pltpu.sync_copy(acc_s.at[sl], accp.at[c, sl])


def _tc1_body(x_ref, w_ref, degp_ref, xw_ref, y_ref):
    deg = jnp.sum(degp_ref[...], axis=0) + 1.0
    dinv = lax.rsqrt(deg)[:, None]
    xw = jnp.dot(x_ref[...], w_ref[...], preferred_element_type=jnp.float32)
    xw_ref[...] = xw
    y_ref[...] = xw * dinv


def _tc2_body(accp_ref, xw_ref, degp_ref, b_ref, w_ref, xw2_ref, y2_ref):
    deg = jnp.sum(degp_ref[...], axis=0) + 1.0
    dinv = lax.rsqrt(deg)[:, None]
    acc = jnp.sum(accp_ref[...], axis=0)
    h = jnp.maximum(acc * dinv + xw_ref[...] * (dinv * dinv) + b_ref[...], 0.0)
    xw2 = jnp.dot(h, w_ref[...], preferred_element_type=jnp.float32)
    xw2_ref[...] = xw2
    y2_ref[...] = xw2 * dinv


def _tc3_body(accp_ref, xw_ref, degp_ref, b_ref, seg_ref, wout_ref, bout_ref,
              out_ref, sums_s, cnt_s):
    i = pl.program_id(0)

    @pl.when(i == 0)
    def _():
        sums_s[...] = jnp.zeros_like(sums_s)
        cnt_s[...] = jnp.zeros_like(cnt_s)

    deg = jnp.sum(degp_ref[...], axis=0) + 1.0
    dinv = lax.rsqrt(deg)[:, None]
    acc = jnp.sum(accp_ref[...], axis=0)
    h = jnp.maximum(acc * dinv + xw_ref[...] * (dinv * dinv) + b_ref[...], 0.0)
    seg = seg_ref[...]
    segid = lax.broadcasted_iota(jnp.int32, (M, BLK), 0)
    onehot_t = jnp.where(segid == seg, 1.0, 0.0)
    sums_s[...] += jnp.dot(onehot_t, h, preferred_element_type=jnp.float32)
    cnt_s[...] += jnp.broadcast_to(
        jnp.sum(onehot_t, axis=1, keepdims=True), (M, D))

    @pl.when(i == pl.num_programs(0) - 1)
    def _():
        z = sums_s[...] / jnp.maximum(cnt_s[...], 1.0)
        out_ref[...] = (
            jnp.dot(z, wout_ref[...], preferred_element_type=jnp.float32)
            + bout_ref[...])


_row_spec = pl.BlockSpec((BLK, D), lambda i: (i, 0))
_w_spec = pl.BlockSpec((D, D), lambda i: (0, 0))
_deg_spec = pl.BlockSpec((NC, BLK), lambda i: (0, i))
_acc_spec = pl.BlockSpec((NC, BLK, D), lambda i: (0, i, 0))
_b_spec = pl.BlockSpec((1, D), lambda i: (0, 0))

_tc1 = pl.pallas_call(
    _tc1_body,
    grid=(GRID,),
    in_specs=[_row_spec, _w_spec, _deg_spec],
    out_specs=[_row_spec, _row_spec],
    out_shape=[jax.ShapeDtypeStruct((NP, D), jnp.float32)] * 2,
)

_tc2 = pl.pallas_call(
    _tc2_body,
    grid=(GRID,),
    in_specs=[_acc_spec, _row_spec, _deg_spec, _b_spec, _w_spec],
    out_specs=[_row_spec, _row_spec],
    out_shape=[jax.ShapeDtypeStruct((NP, D), jnp.float32)] * 2,
)

_tc3 = pl.pallas_call(
    _tc3_body,
    grid=(GRID,),
    in_specs=[_acc_spec, _row_spec, _deg_spec, _b_spec,
              pl.BlockSpec((1, BLK), lambda i: (0, i)), _w_spec, _b_spec],
    out_specs=pl.BlockSpec((M, D), lambda i: (0, 0)),
    out_shape=jax.ShapeDtypeStruct((M, D), jnp.float32),
    scratch_shapes=[pltpu.VMEM((M, D), jnp.float32),
                    pltpu.VMEM((M, D), jnp.float32)],
)


def kernel(x, edge_index, batch_idx, W1, b1, W2, b2, Wout, bout):
    pad = jnp.full((EP - E,), TRASH, dtype=jnp.int32)
    src = jnp.concatenate([edge_index[0], pad])
    dst = jnp.concatenate([edge_index[1], pad])
    srcr = src.reshape(NW, NB, C)
    dstr = dst.reshape(NW, NB, C)
    dsth = dst.reshape(NW, RD, C)
    x_pad = jnp.pad(x, ((0, NP - N), (0, 0)))
    seg_pad = jnp.pad(batch_idx, (0, NP - N), constant_values=M).reshape(1, NP)
    b1r = b1.reshape(1, D)
    b2r = b2.reshape(1, D)
    boutr = bout.reshape(1, D)

    degp = _deg_kernel(dsth).reshape(NC, NP)
    xw1, y1 = _tc1(x_pad, W1, degp)
    acc1 = _prop_kernel(y1, srcr, dstr)
    xw2, y2 = _tc2(acc1, xw1, degp, b1r, W2)
    acc2 = _prop_kernel(y2, srcr, dstr)
    return _tc3(acc2, xw2, degp, b2r, seg_pad, Wout, boutr)


# speedup vs baseline: 1.0975x; 1.0975x over previous
import functools

import jax
import jax.numpy as jnp
from jax import lax
from jax.experimental import pallas as pl
from jax.experimental.pallas import tpu as pltpu
from jax.experimental.pallas import tpu_sc as plsc

N = 10000
NP = 10240
E = 320000
D = 128
M = 64

NC = 2
NS = 16
NW = NC * NS
C = 128
EP = 327680
RD = EP // NW // C
NB0 = 56
NB1 = (EP // C - NS * NB0) // NS
NBM = max(NB0, NB1)
RPT = NP // NS
TRASH = NP - 1
BLK = 2048
GRID = NP // BLK

_sc_mesh = plsc.VectorSubcoreMesh(
    core_axis_name="c", subcore_axis_name="s", num_cores=NC, num_subcores=NS)
_sc_params = pltpu.CompilerParams(needs_layout_passes=False)

_zero16 = functools.partial(jnp.zeros, (16,), jnp.float32)


@functools.partial(
    pl.kernel,
    out_type=jax.ShapeDtypeStruct((NC * NP,), jnp.float32),
    mesh=_sc_mesh,
    scratch_types=[
        pltpu.VMEM((RD, C), jnp.int32),
        pltpu.VMEM((NP,), jnp.float32),
        pltpu.VMEM((RPT,), jnp.float32),
        pltpu.VMEM((RPT,), jnp.float32),
        pltpu.VMEM_SHARED((NS * NP,), jnp.float32),
    ],
    compiler_params=_sc_params,
)
def _deg_kernel(dst_hbm, deg_out, idx_v, hist_v, tmp_v, acc_v, shared_s):
    c = lax.axis_index("c")
    s = lax.axis_index("s")
    pltpu.sync_copy(dst_hbm.at[c * NS + s], idx_v)

    def _zero_hist(i, _):
        hist_v[pl.ds(i * 16, 16)] = _zero16()
        return 0
    lax.fori_loop(0, NP // 16, _zero_hist, 0)

    ones = jnp.ones((16,), jnp.float32)

    def _count(r, _):
        for k in range(C // 16):
            plsc.addupdate_scatter(hist_v, [idx_v[r, pl.ds(k * 16, 16)]], ones)
        return 0
    lax.fori_loop(0, RD, _count, 0)

    pltpu.sync_copy(hist_v, shared_s.at[pl.ds(s * NP, NP)])
    plsc.subcore_barrier()

    base = s * RPT

    def _zero_acc(i, _):
        acc_v[pl.ds(i * 16, 16)] = _zero16()
        return 0
    lax.fori_loop(0, RPT // 16, _zero_acc, 0)

    for k in range(NS):
        pltpu.sync_copy(shared_s.at[pl.ds(k * NP + base, RPT)], tmp_v)

        def _addv(i, _):
            acc_v[pl.ds(i * 16, 16)] = (
                acc_v[pl.ds(i * 16, 16)] + tmp_v[pl.ds(i * 16, 16)])
            return 0
        lax.fori_loop(0, RPT // 16, _addv, 0)

    pltpu.sync_copy(acc_v, deg_out.at[pl.ds(c * NP + base, RPT)])


@functools.partial(
    pl.kernel,
    out_type=jax.ShapeDtypeStruct((NC, NP, D), jnp.float32),
    mesh=_sc_mesh,
    scratch_types=[
        pltpu.VMEM((NBM, C), jnp.int32),
        pltpu.VMEM((NBM, C), jnp.int32),
        pltpu.VMEM((C, D), jnp.float32),
        pltpu.VMEM_SHARED((NP, D), jnp.float32),
        pltpu.SemaphoreType.DMA,
    ],
    compiler_params=_sc_params,
)
def _prop_kernel(y_hbm, srcr, dstr, accp, sidx_v, didx_v, rows_v, acc_s, sem):
    c = lax.axis_index("c")
    s = lax.axis_index("s")

    @pl.when(c == 0)
    def _():
        base0 = s * NB0
        pltpu.sync_copy(srcr.at[pl.ds(base0, NB0)], sidx_v.at[pl.ds(0, NB0)])
        pltpu.sync_copy(dstr.at[pl.ds(base0, NB0)], didx_v.at[pl.ds(0, NB0)])

    @pl.when(c == 1)
    def _():
        base1 = NS * NB0 + s * NB1
        pltpu.sync_copy(srcr.at[pl.ds(base1, NB1)], sidx_v.at[pl.ds(0, NB1)])
        pltpu.sync_copy(dstr.at[pl.ds(base1, NB1)], didx_v.at[pl.ds(0, NB1)])

    def _zrow(i, _):
        for k in range(D // 16):
            rows_v[i, pl.ds(k * 16, 16)] = _zero16()
        return 0
    lax.fori_loop(0, C, _zrow, 0)

    base = s * RPT
    for t in range(RPT // C):
        pltpu.sync_copy(rows_v, acc_s.at[pl.ds(base + t * C, C)])
    plsc.subcore_barrier()

    nb = jnp.where(c == 0, NB0, NB1)

    def _blk(b, _):
        pltpu.async_copy(y_hbm.at[sidx_v.at[b]], rows_v, sem).wait()
        pltpu.sync_copy(rows_v, acc_s.at[didx_v.at[b]], add=True)
        return 0
    lax.fori_loop(0, nb, _blk, 0)

    plsc.subcore_barrier()
    for t in range(RPT // C):
        sl = pl.ds(base + t * C, C)
        pltpu.sync_copy(acc_s.at[sl], accp.at[c, sl])


def _tc1_body(x_ref, w_ref, degp_ref, xw_ref, y_ref):
    deg = jnp.sum(degp_ref[...], axis=0) + 1.0
    dinv = lax.rsqrt(deg)[:, None]
    xw = jnp.dot(x_ref[...], w_ref[...], preferred_element_type=jnp.float32)
    xw_ref[...] = xw
    y_ref[...] = xw * dinv


def _tc2_body(accp_ref, xw_ref, degp_ref, b_ref, w_ref, xw2_ref, y2_ref):
    deg = jnp.sum(degp_ref[...], axis=0) + 1.0
    dinv = lax.rsqrt(deg)[:, None]
    acc = jnp.sum(accp_ref[...], axis=0)
    h = jnp.maximum(acc * dinv + xw_ref[...] * (dinv * dinv) + b_ref[...], 0.0)
    xw2 = jnp.dot(h, w_ref[...], preferred_element_type=jnp.float32)
    xw2_ref[...] = xw2
    y2_ref[...] = xw2 * dinv


def _tc3_body(accp_ref, xw_ref, degp_ref, b_ref, seg_ref, wout_ref, bout_ref,
              out_ref, sums_s, cnt_s):
    i = pl.program_id(0)

    @pl.when(i == 0)
    def _():
        sums_s[...] = jnp.zeros_like(sums_s)
        cnt_s[...] = jnp.zeros_like(cnt_s)

    deg = jnp.sum(degp_ref[...], axis=0) + 1.0
    dinv = lax.rsqrt(deg)[:, None]
    acc = jnp.sum(accp_ref[...], axis=0)
    h = jnp.maximum(acc * dinv + xw_ref[...] * (dinv * dinv) + b_ref[...], 0.0)
    seg = seg_ref[...]
    segid = lax.broadcasted_iota(jnp.int32, (M, BLK), 0)
    onehot_t = jnp.where(segid == seg, 1.0, 0.0)
    sums_s[...] += jnp.dot(onehot_t, h, preferred_element_type=jnp.float32)
    cnt_s[...] += jnp.broadcast_to(
        jnp.sum(onehot_t, axis=1, keepdims=True), (M, D))

    @pl.when(i == pl.num_programs(0) - 1)
    def _():
        z = sums_s[...] / jnp.maximum(cnt_s[...], 1.0)
        out_ref[...] = (
            jnp.dot(z, wout_ref[...], preferred_element_type=jnp.float32)
            + bout_ref[...])


_row_spec = pl.BlockSpec((BLK, D), lambda i: (i, 0))
_w_spec = pl.BlockSpec((D, D), lambda i: (0, 0))
_deg_spec = pl.BlockSpec((NC, BLK), lambda i: (0, i))
_acc_spec = pl.BlockSpec((NC, BLK, D), lambda i: (0, i, 0))
_b_spec = pl.BlockSpec((1, D), lambda i: (0, 0))

_tc1 = pl.pallas_call(
    _tc1_body,
    grid=(GRID,),
    in_specs=[_row_spec, _w_spec, _deg_spec],
    out_specs=[_row_spec, _row_spec],
    out_shape=[jax.ShapeDtypeStruct((NP, D), jnp.float32)] * 2,
)

_tc2 = pl.pallas_call(
    _tc2_body,
    grid=(GRID,),
    in_specs=[_acc_spec, _row_spec, _deg_spec, _b_spec, _w_spec],
    out_specs=[_row_spec, _row_spec],
    out_shape=[jax.ShapeDtypeStruct((NP, D), jnp.float32)] * 2,
)

_tc3 = pl.pallas_call(
    _tc3_body,
    grid=(GRID,),
    in_specs=[_acc_spec, _row_spec, _deg_spec, _b_spec,
              pl.BlockSpec((1, BLK), lambda i: (0, i)), _w_spec, _b_spec],
    out_specs=pl.BlockSpec((M, D), lambda i: (0, 0)),
    out_shape=jax.ShapeDtypeStruct((M, D), jnp.float32),
    scratch_shapes=[pltpu.VMEM((M, D), jnp.float32),
                    pltpu.VMEM((M, D), jnp.float32)],
)


def kernel(x, edge_index, batch_idx, W1, b1, W2, b2, Wout, bout):
    pad = jnp.full((EP - E,), TRASH, dtype=jnp.int32)
    src = jnp.concatenate([edge_index[0], pad])
    dst = jnp.concatenate([edge_index[1], pad])
    srcr = src.reshape(EP // C, C)
    dstr = dst.reshape(EP // C, C)
    dsth = dst.reshape(NW, RD, C)
    x_pad = jnp.pad(x, ((0, NP - N), (0, 0)))
    seg_pad = jnp.pad(batch_idx, (0, NP - N), constant_values=M).reshape(1, NP)
    b1r = b1.reshape(1, D)
    b2r = b2.reshape(1, D)
    boutr = bout.reshape(1, D)

    degp = _deg_kernel(dsth).reshape(NC, NP)
    xw1, y1 = _tc1(x_pad, W1, degp)
    acc1 = _prop_kernel(y1, srcr, dstr)
    xw2, y2 = _tc2(acc1, xw1, degp, b1r, W2)
    acc2 = _prop_kernel(y2, srcr, dstr)
    return _tc3(acc2, xw2, degp, b2r, seg_pad, Wout, boutr)


# speedup vs baseline: 1.3253x; 1.2076x over previous
import functools

import jax
import jax.numpy as jnp
from jax import lax
from jax.experimental import pallas as pl
from jax.experimental.pallas import tpu as pltpu
from jax.experimental.pallas import tpu_sc as plsc

N = 10000
NP = 10240
E = 320000
D = 128
M = 64

NC = 2
NS = 16
NW = NC * NS
C = 128
EP = 327680
RD = EP // NW // C
NB0 = 112
NB1 = (EP // C - NS * NB0) // NS
NBM = max(NB0, NB1)
RPT = NP // NS
TRASH = NP - 1
BLK = 2048
GRID = NP // BLK

_sc_mesh = plsc.VectorSubcoreMesh(
    core_axis_name="c", subcore_axis_name="s", num_cores=NC, num_subcores=NS)
_sc_params = pltpu.CompilerParams(needs_layout_passes=False)

_zero16 = functools.partial(jnp.zeros, (16,), jnp.float32)


@functools.partial(
    pl.kernel,
    out_type=jax.ShapeDtypeStruct((NC * NP,), jnp.float32),
    mesh=_sc_mesh,
    scratch_types=[
        pltpu.VMEM((RD, C), jnp.int32),
        pltpu.VMEM((NP,), jnp.float32),
        pltpu.VMEM((RPT,), jnp.float32),
        pltpu.VMEM((RPT,), jnp.float32),
        pltpu.VMEM_SHARED((NS * NP,), jnp.float32),
    ],
    compiler_params=_sc_params,
)
def _deg_kernel(dst_hbm, deg_out, idx_v, hist_v, tmp_v, acc_v, shared_s):
    c = lax.axis_index("c")
    s = lax.axis_index("s")
    pltpu.sync_copy(dst_hbm.at[c * NS + s], idx_v)

    def _zero_hist(i, _):
        hist_v[pl.ds(i * 16, 16)] = _zero16()
        return 0
    lax.fori_loop(0, NP // 16, _zero_hist, 0)

    ones = jnp.ones((16,), jnp.float32)

    def _count(r, _):
        for k in range(C // 16):
            plsc.addupdate_scatter(hist_v, [idx_v[r, pl.ds(k * 16, 16)]], ones)
        return 0
    lax.fori_loop(0, RD, _count, 0)

    pltpu.sync_copy(hist_v, shared_s.at[pl.ds(s * NP, NP)])
    plsc.subcore_barrier()

    base = s * RPT

    def _zero_acc(i, _):
        acc_v[pl.ds(i * 16, 16)] = _zero16()
        return 0
    lax.fori_loop(0, RPT // 16, _zero_acc, 0)

    for k in range(NS):
        pltpu.sync_copy(shared_s.at[pl.ds(k * NP + base, RPT)], tmp_v)

        def _addv(i, _):
            acc_v[pl.ds(i * 16, 16)] = (
                acc_v[pl.ds(i * 16, 16)] + tmp_v[pl.ds(i * 16, 16)])
            return 0
        lax.fori_loop(0, RPT // 16, _addv, 0)

    pltpu.sync_copy(acc_v, deg_out.at[pl.ds(c * NP + base, RPT)])


@functools.partial(
    pl.kernel,
    out_type=jax.ShapeDtypeStruct((NC, NP, D), jnp.float32),
    mesh=_sc_mesh,
    scratch_types=[
        pltpu.VMEM((NBM, C), jnp.int32),
        pltpu.VMEM((NBM, C), jnp.int32),
        pltpu.VMEM((C, D), jnp.float32),
        pltpu.VMEM_SHARED((NP, D), jnp.float32),
        pltpu.SemaphoreType.DMA,
    ],
    compiler_params=_sc_params,
)
def _prop_kernel(y_hbm, srcr, dstr, accp, sidx_v, didx_v, rows_v, acc_s, sem):
    c = lax.axis_index("c")
    s = lax.axis_index("s")

    @pl.when(c == 0)
    def _():
        base0 = s * NB0
        pltpu.sync_copy(srcr.at[pl.ds(base0, NB0)], sidx_v.at[pl.ds(0, NB0)])
        pltpu.sync_copy(dstr.at[pl.ds(base0, NB0)], didx_v.at[pl.ds(0, NB0)])

    @pl.when(c == 1)
    def _():
        base1 = NS * NB0 + s * NB1
        pltpu.sync_copy(srcr.at[pl.ds(base1, NB1)], sidx_v.at[pl.ds(0, NB1)])
        pltpu.sync_copy(dstr.at[pl.ds(base1, NB1)], didx_v.at[pl.ds(0, NB1)])

    def _zrow(i, _):
        for k in range(D // 16):
            rows_v[i, pl.ds(k * 16, 16)] = _zero16()
        return 0
    lax.fori_loop(0, C, _zrow, 0)

    base = s * RPT
    for t in range(RPT // C):
        pltpu.sync_copy(rows_v, acc_s.at[pl.ds(base + t * C, C)])
    plsc.subcore_barrier()

    nb = jnp.where(c == 0, NB0, NB1)

    def _blk(b, _):
        pltpu.async_copy(y_hbm.at[sidx_v.at[b]], rows_v, sem).wait()
        pltpu.sync_copy(rows_v, acc_s.at[didx_v.at[b]], add=True)
        return 0
    lax.fori_loop(0, nb, _blk, 0)

    plsc.subcore_barrier()
    for t in range(RPT // C):
        sl = pl.ds(base + t * C, C)
        pltpu.sync_copy(acc_s.at[sl], accp.at[c, sl])


def _tc1_body(x_ref, w_ref, degp_ref, xw_ref, y_ref):
    deg = jnp.sum(degp_ref[...], axis=0) + 1.0
    dinv = lax.rsqrt(deg)[:, None]
    xw = jnp.dot(x_ref[...], w_ref[...], preferred_element_type=jnp.float32)
    xw_ref[...] = xw
    y_ref[...] = xw * dinv


def _tc2_body(accp_ref, xw_ref, degp_ref, b_ref, w_ref, xw2_ref, y2_ref):
    deg = jnp.sum(degp_ref[...], axis=0) + 1.0
    dinv = lax.rsqrt(deg)[:, None]
    acc = jnp.sum(accp_ref[...], axis=0)
    h = jnp.maximum(acc * dinv + xw_ref[...] * (dinv * dinv) + b_ref[...], 0.0)
    xw2 = jnp.dot(h, w_ref[...], preferred_element_type=jnp.float32)
    xw2_ref[...] = xw2
    y2_ref[...] = xw2 * dinv


def _tc3_body(accp_ref, xw_ref, degp_ref, b_ref, seg_ref, wout_ref, bout_ref,
              out_ref, sums_s, cnt_s):
    i = pl.program_id(0)

    @pl.when(i == 0)
    def _():
        sums_s[...] = jnp.zeros_like(sums_s)
        cnt_s[...] = jnp.zeros_like(cnt_s)

    deg = jnp.sum(degp_ref[...], axis=0) + 1.0
    dinv = lax.rsqrt(deg)[:, None]
    acc = jnp.sum(accp_ref[...], axis=0)
    h = jnp.maximum(acc * dinv + xw_ref[...] * (dinv * dinv) + b_ref[...], 0.0)
    seg = seg_ref[...]
    segid = lax.broadcasted_iota(jnp.int32, (M, BLK), 0)
    onehot_t = jnp.where(segid == seg, 1.0, 0.0)
    sums_s[...] += jnp.dot(onehot_t, h, preferred_element_type=jnp.float32)
    cnt_s[...] += jnp.broadcast_to(
        jnp.sum(onehot_t, axis=1, keepdims=True), (M, D))

    @pl.when(i == pl.num_programs(0) - 1)
    def _():
        z = sums_s[...] / jnp.maximum(cnt_s[...], 1.0)
        out_ref[...] = (
            jnp.dot(z, wout_ref[...], preferred_element_type=jnp.float32)
            + bout_ref[...])


_row_spec = pl.BlockSpec((BLK, D), lambda i: (i, 0))
_w_spec = pl.BlockSpec((D, D), lambda i: (0, 0))
_deg_spec = pl.BlockSpec((NC, BLK), lambda i: (0, i))
_acc_spec = pl.BlockSpec((NC, BLK, D), lambda i: (0, i, 0))
_b_spec = pl.BlockSpec((1, D), lambda i: (0, 0))

_tc1 = pl.pallas_call(
    _tc1_body,
    grid=(GRID,),
    in_specs=[_row_spec, _w_spec, _deg_spec],
    out_specs=[_row_spec, _row_spec],
    out_shape=[jax.ShapeDtypeStruct((NP, D), jnp.float32)] * 2,
)

_tc2 = pl.pallas_call(
    _tc2_body,
    grid=(GRID,),
    in_specs=[_acc_spec, _row_spec, _deg_spec, _b_spec, _w_spec],
    out_specs=[_row_spec, _row_spec],
    out_shape=[jax.ShapeDtypeStruct((NP, D), jnp.float32)] * 2,
)

_tc3 = pl.pallas_call(
    _tc3_body,
    grid=(GRID,),
    in_specs=[_acc_spec, _row_spec, _deg_spec, _b_spec,
              pl.BlockSpec((1, BLK), lambda i: (0, i)), _w_spec, _b_spec],
    out_specs=pl.BlockSpec((M, D), lambda i: (0, 0)),
    out_shape=jax.ShapeDtypeStruct((M, D), jnp.float32),
    scratch_shapes=[pltpu.VMEM((M, D), jnp.float32),
                    pltpu.VMEM((M, D), jnp.float32)],
)


def kernel(x, edge_index, batch_idx, W1, b1, W2, b2, Wout, bout):
    pad = jnp.full((EP - E,), TRASH, dtype=jnp.int32)
    src = jnp.concatenate([edge_index[0], pad])
    dst = jnp.concatenate([edge_index[1], pad])
    srcr = src.reshape(EP // C, C)
    dstr = dst.reshape(EP // C, C)
    dsth = dst.reshape(NW, RD, C)
    x_pad = jnp.pad(x, ((0, NP - N), (0, 0)))
    seg_pad = jnp.pad(batch_idx, (0, NP - N), constant_values=M).reshape(1, NP)
    b1r = b1.reshape(1, D)
    b2r = b2.reshape(1, D)
    boutr = bout.reshape(1, D)

    degp = _deg_kernel(dsth).reshape(NC, NP)
    xw1, y1 = _tc1(x_pad, W1, degp)
    acc1 = _prop_kernel(y1, srcr, dstr)
    xw2, y2 = _tc2(acc1, xw1, degp, b1r, W2)
    acc2 = _prop_kernel(y2, srcr, dstr)
    return _tc3(acc2, xw2, degp, b2r, seg_pad, Wout, boutr)


# speedup vs baseline: 2.8273x; 2.1333x over previous
import functools

import jax
import jax.numpy as jnp
from jax import lax
from jax.experimental import pallas as pl
from jax.experimental.pallas import tpu as pltpu
from jax.experimental.pallas import tpu_sc as plsc

N = 10000
NP = 10240
E = 320000
D = 128
M = 64

NC = 2
NS = 16
NW = NC * NS
C = 128
EP = 327680
RD = EP // NW // C
NB0 = 80
NB1 = (EP // C - NS * NB0) // NS
NBM = max(NB0, NB1)
RPT = NP // NS
TRASH = NP - 1
BLK = 2048
GRID = NP // BLK

_sc_mesh = plsc.VectorSubcoreMesh(
    core_axis_name="c", subcore_axis_name="s", num_cores=NC, num_subcores=NS)
_sc_params = pltpu.CompilerParams(needs_layout_passes=False)

_zero16 = functools.partial(jnp.zeros, (16,), jnp.float32)


@functools.partial(
    pl.kernel,
    out_type=jax.ShapeDtypeStruct((NC * NP,), jnp.float32),
    mesh=_sc_mesh,
    scratch_types=[
        pltpu.VMEM((RD, C), jnp.int32),
        pltpu.VMEM((NP,), jnp.float32),
        pltpu.VMEM((RPT,), jnp.float32),
        pltpu.VMEM((RPT,), jnp.float32),
        pltpu.VMEM_SHARED((NS * NP,), jnp.float32),
    ],
    compiler_params=_sc_params,
)
def _deg_kernel(dst_hbm, deg_out, idx_v, hist_v, tmp_v, acc_v, shared_s):
    c = lax.axis_index("c")
    s = lax.axis_index("s")
    pltpu.sync_copy(dst_hbm.at[c * NS + s], idx_v)

    def _zero_hist(i, _):
        hist_v[pl.ds(i * 16, 16)] = _zero16()
        return 0
    lax.fori_loop(0, NP // 16, _zero_hist, 0)

    ones = jnp.ones((16,), jnp.float32)

    def _count(r, _):
        for k in range(C // 16):
            plsc.addupdate_scatter(hist_v, [idx_v[r, pl.ds(k * 16, 16)]], ones)
        return 0
    lax.fori_loop(0, RD, _count, 0)

    pltpu.sync_copy(hist_v, shared_s.at[pl.ds(s * NP, NP)])
    plsc.subcore_barrier()

    base = s * RPT

    def _zero_acc(i, _):
        acc_v[pl.ds(i * 16, 16)] = _zero16()
        return 0
    lax.fori_loop(0, RPT // 16, _zero_acc, 0)

    for k in range(NS):
        pltpu.sync_copy(shared_s.at[pl.ds(k * NP + base, RPT)], tmp_v)

        def _addv(i, _):
            acc_v[pl.ds(i * 16, 16)] = (
                acc_v[pl.ds(i * 16, 16)] + tmp_v[pl.ds(i * 16, 16)])
            return 0
        lax.fori_loop(0, RPT // 16, _addv, 0)

    pltpu.sync_copy(acc_v, deg_out.at[pl.ds(c * NP + base, RPT)])


@functools.partial(
    pl.kernel,
    out_type=jax.ShapeDtypeStruct((NC, NP, D), jnp.float32),
    mesh=_sc_mesh,
    scratch_types=[
        pltpu.VMEM((NBM, C), jnp.int32),
        pltpu.VMEM((NBM, C), jnp.int32),
        pltpu.VMEM((C, D), jnp.float32),
        pltpu.VMEM_SHARED((NP, D), jnp.float32),
        pltpu.SemaphoreType.DMA,
    ],
    compiler_params=_sc_params,
)
def _prop_kernel(y_hbm, srcr, dstr, accp, sidx_v, didx_v, rows_v, acc_s, sem):
    c = lax.axis_index("c")
    s = lax.axis_index("s")

    @pl.when(c == 0)
    def _():
        base0 = s * NB0
        pltpu.sync_copy(srcr.at[pl.ds(base0, NB0)], sidx_v.at[pl.ds(0, NB0)])
        pltpu.sync_copy(dstr.at[pl.ds(base0, NB0)], didx_v.at[pl.ds(0, NB0)])

    @pl.when(c == 1)
    def _():
        base1 = NS * NB0 + s * NB1
        pltpu.sync_copy(srcr.at[pl.ds(base1, NB1)], sidx_v.at[pl.ds(0, NB1)])
        pltpu.sync_copy(dstr.at[pl.ds(base1, NB1)], didx_v.at[pl.ds(0, NB1)])

    def _zrow(i, _):
        for k in range(D // 16):
            rows_v[i, pl.ds(k * 16, 16)] = _zero16()
        return 0
    lax.fori_loop(0, C, _zrow, 0)

    base = s * RPT
    for t in range(RPT // C):
        pltpu.sync_copy(rows_v, acc_s.at[pl.ds(base + t * C, C)])
    plsc.subcore_barrier()

    nb = jnp.where(c == 0, NB0, NB1)

    def _blk(b, _):
        pltpu.async_copy(y_hbm.at[sidx_v.at[b]], rows_v, sem).wait()
        pltpu.sync_copy(rows_v, acc_s.at[didx_v.at[b]], add=True)
        return 0
    lax.fori_loop(0, nb, _blk, 0)

    plsc.subcore_barrier()
    for t in range(RPT // C):
        sl = pl.ds(base + t * C, C)
        pltpu.sync_copy(acc_s.at[sl], accp.at[c, sl])


def _tc1_body(x_ref, w_ref, degp_ref, xw_ref, y_ref):
    deg = jnp.sum(degp_ref[...], axis=0) + 1.0
    dinv = lax.rsqrt(deg)[:, None]
    xw = jnp.dot(x_ref[...], w_ref[...], preferred_element_type=jnp.float32)
    xw_ref[...] = xw
    y_ref[...] = xw * dinv


def _tc2_body(accp_ref, xw_ref, degp_ref, b_ref, w_ref, xw2_ref, y2_ref):
    deg = jnp.sum(degp_ref[...], axis=0) + 1.0
    dinv = lax.rsqrt(deg)[:, None]
    acc = jnp.sum(accp_ref[...], axis=0)
    h = jnp.maximum(acc * dinv + xw_ref[...] * (dinv * dinv) + b_ref[...], 0.0)
    xw2 = jnp.dot(h, w_ref[...], preferred_element_type=jnp.float32)
    xw2_ref[...] = xw2
    y2_ref[...] = xw2 * dinv


def _tc3_body(accp_ref, xw_ref, degp_ref, b_ref, seg_ref, wout_ref, bout_ref,
              out_ref, sums_s, cnt_s):
    i = pl.program_id(0)

    @pl.when(i == 0)
    def _():
        sums_s[...] = jnp.zeros_like(sums_s)
        cnt_s[...] = jnp.zeros_like(cnt_s)

    deg = jnp.sum(degp_ref[...], axis=0) + 1.0
    dinv = lax.rsqrt(deg)[:, None]
    acc = jnp.sum(accp_ref[...], axis=0)
    h = jnp.maximum(acc * dinv + xw_ref[...] * (dinv * dinv) + b_ref[...], 0.0)
    seg = seg_ref[...]
    segid = lax.broadcasted_iota(jnp.int32, (M, BLK), 0)
    onehot_t = jnp.where(segid == seg, 1.0, 0.0)
    sums_s[...] += jnp.dot(onehot_t, h, preferred_element_type=jnp.float32)
    cnt_s[...] += jnp.broadcast_to(
        jnp.sum(onehot_t, axis=1, keepdims=True), (M, D))

    @pl.when(i == pl.num_programs(0) - 1)
    def _():
        z = sums_s[...] / jnp.maximum(cnt_s[...], 1.0)
        out_ref[...] = (
            jnp.dot(z, wout_ref[...], preferred_element_type=jnp.float32)
            + bout_ref[...])


_row_spec = pl.BlockSpec((BLK, D), lambda i: (i, 0))
_w_spec = pl.BlockSpec((D, D), lambda i: (0, 0))
_deg_spec = pl.BlockSpec((NC, BLK), lambda i: (0, i))
_acc_spec = pl.BlockSpec((NC, BLK, D), lambda i: (0, i, 0))
_b_spec = pl.BlockSpec((1, D), lambda i: (0, 0))

_tc1 = pl.pallas_call(
    _tc1_body,
    grid=(GRID,),
    in_specs=[_row_spec, _w_spec, _deg_spec],
    out_specs=[_row_spec, _row_spec],
    out_shape=[jax.ShapeDtypeStruct((NP, D), jnp.float32)] * 2,
)

_tc2 = pl.pallas_call(
    _tc2_body,
    grid=(GRID,),
    in_specs=[_acc_spec, _row_spec, _deg_spec, _b_spec, _w_spec],
    out_specs=[_row_spec, _row_spec],
    out_shape=[jax.ShapeDtypeStruct((NP, D), jnp.float32)] * 2,
)

_tc3 = pl.pallas_call(
    _tc3_body,
    grid=(GRID,),
    in_specs=[_acc_spec, _row_spec, _deg_spec, _b_spec,
              pl.BlockSpec((1, BLK), lambda i: (0, i)), _w_spec, _b_spec],
    out_specs=pl.BlockSpec((M, D), lambda i: (0, 0)),
    out_shape=jax.ShapeDtypeStruct((M, D), jnp.float32),
    scratch_shapes=[pltpu.VMEM((M, D), jnp.float32),
                    pltpu.VMEM((M, D), jnp.float32)],
)


def kernel(x, edge_index, batch_idx, W1, b1, W2, b2, Wout, bout):
    pad = N + (jnp.arange(EP - E, dtype=jnp.int32) % (NP - N))
    src = jnp.concatenate([edge_index[0], pad])
    dst = jnp.concatenate([edge_index[1], pad])
    srcr = src.reshape(EP // C, C)
    dstr = dst.reshape(EP // C, C)
    dsth = dst.reshape(NW, RD, C)
    x_pad = jnp.pad(x, ((0, NP - N), (0, 0)))
    seg_pad = jnp.pad(batch_idx, (0, NP - N), constant_values=M).reshape(1, NP)
    b1r = b1.reshape(1, D)
    b2r = b2.reshape(1, D)
    boutr = bout.reshape(1, D)

    degp = _deg_kernel(dsth).reshape(NC, NP)
    xw1, y1 = _tc1(x_pad, W1, degp)
    acc1 = _prop_kernel(y1, srcr, dstr)
    xw2, y2 = _tc2(acc1, xw1, degp, b1r, W2)
    acc2 = _prop_kernel(y2, srcr, dstr)
    return _tc3(acc2, xw2, degp, b2r, seg_pad, Wout, boutr)


# speedup vs baseline: 3.5712x; 1.2631x over previous
import functools

import jax
import jax.numpy as jnp
from jax import lax
from jax.experimental import pallas as pl
from jax.experimental.pallas import tpu as pltpu
from jax.experimental.pallas import tpu_sc as plsc

N = 10000
NP = 10240
E = 320000
D = 128
M = 64

NC = 2
NS = 16
NW = NC * NS
C = 128
EP = 327680
RD = EP // NW // C
NB = EP // NW // C
CH = 16
NCH = NB // CH
RPT = NP // NS
TRASH = NP - 1
BLK = 2048
GRID = NP // BLK

_sc_mesh = plsc.VectorSubcoreMesh(
    core_axis_name="c", subcore_axis_name="s", num_cores=NC, num_subcores=NS)
_sc_params = pltpu.CompilerParams(needs_layout_passes=False)

_zero16 = functools.partial(jnp.zeros, (16,), jnp.float32)


@functools.partial(
    pl.kernel,
    out_type=jax.ShapeDtypeStruct((NC * NP,), jnp.float32),
    mesh=_sc_mesh,
    scratch_types=[
        pltpu.VMEM((RD, C), jnp.int32),
        pltpu.VMEM((NP,), jnp.float32),
        pltpu.VMEM((RPT,), jnp.float32),
        pltpu.VMEM((RPT,), jnp.float32),
        pltpu.VMEM_SHARED((NS * NP,), jnp.float32),
    ],
    compiler_params=_sc_params,
)
def _deg_kernel(dst_hbm, deg_out, idx_v, hist_v, tmp_v, acc_v, shared_s):
    c = lax.axis_index("c")
    s = lax.axis_index("s")
    pltpu.sync_copy(dst_hbm.at[c * NS + s], idx_v)

    def _zero_hist(i, _):
        hist_v[pl.ds(i * 16, 16)] = _zero16()
        return 0
    lax.fori_loop(0, NP // 16, _zero_hist, 0)

    ones = jnp.ones((16,), jnp.float32)

    def _count(r, _):
        for k in range(C // 16):
            plsc.addupdate_scatter(hist_v, [idx_v[r, pl.ds(k * 16, 16)]], ones)
        return 0
    lax.fori_loop(0, RD, _count, 0)

    pltpu.sync_copy(hist_v, shared_s.at[pl.ds(s * NP, NP)])
    plsc.subcore_barrier()

    base = s * RPT

    def _zero_acc(i, _):
        acc_v[pl.ds(i * 16, 16)] = _zero16()
        return 0
    lax.fori_loop(0, RPT // 16, _zero_acc, 0)

    for k in range(NS):
        pltpu.sync_copy(shared_s.at[pl.ds(k * NP + base, RPT)], tmp_v)

        def _addv(i, _):
            acc_v[pl.ds(i * 16, 16)] = (
                acc_v[pl.ds(i * 16, 16)] + tmp_v[pl.ds(i * 16, 16)])
            return 0
        lax.fori_loop(0, RPT // 16, _addv, 0)

    pltpu.sync_copy(acc_v, deg_out.at[pl.ds(c * NP + base, RPT)])


@functools.partial(
    pl.kernel,
    out_type=jax.ShapeDtypeStruct((NC, NP, D), jnp.float32),
    mesh=_sc_mesh,
    scratch_types=[
        pltpu.VMEM((2, CH, C), jnp.int32),
        pltpu.VMEM((2, CH, C), jnp.int32),
        pltpu.VMEM((C, D), jnp.float32),
        pltpu.VMEM((C, D), jnp.float32),
        pltpu.VMEM_SHARED((NP, D), jnp.float32),
        pltpu.SemaphoreType.DMA,
        pltpu.SemaphoreType.DMA,
        pltpu.SemaphoreType.DMA,
        pltpu.SemaphoreType.DMA,
    ],
    compiler_params=_sc_params,
)
def _prop_kernel(y_hbm, srcr, dstr, accp, sidx_v, didx_v, rows_a, rows_b,
                 acc_s, gsem_a, gsem_b, isem_s, isem_d):
    c = lax.axis_index("c")
    s = lax.axis_index("s")
    wid = c * NS + s

    def _zrow(i, _):
        for k in range(D // 16):
            rows_a[i, pl.ds(k * 16, 16)] = _zero16()
        return 0
    lax.fori_loop(0, C, _zrow, 0)

    base = s * RPT
    for t in range(RPT // C):
        pltpu.sync_copy(rows_a, acc_s.at[pl.ds(base + t * C, C)])

    pltpu.sync_copy(srcr.at[wid, pl.ds(0, CH)], sidx_v.at[0])
    pltpu.sync_copy(dstr.at[wid, pl.ds(0, CH)], didx_v.at[0])
    pltpu.async_copy(y_hbm.at[sidx_v.at[0, 0]], rows_a, gsem_a)
    plsc.subcore_barrier()

    for ci in range(NCH):
        p = ci % 2
        nxt = ci + 1
        if nxt < NCH:
            pltpu.async_copy(srcr.at[wid, pl.ds(nxt * CH, CH)],
                             sidx_v.at[nxt % 2], isem_s)
            pltpu.async_copy(dstr.at[wid, pl.ds(nxt * CH, CH)],
                             didx_v.at[nxt % 2], isem_d)

        def _pair(j, _):
            ra = 2 * j
            rb = 2 * j + 1
            pltpu.make_async_copy(
                y_hbm.at[sidx_v.at[p, ra]], rows_a, gsem_a).wait()
            pltpu.async_copy(y_hbm.at[sidx_v.at[p, rb]], rows_b, gsem_b)
            pltpu.sync_copy(rows_a, acc_s.at[didx_v.at[p, ra]], add=True)
            pltpu.make_async_copy(
                y_hbm.at[sidx_v.at[p, rb]], rows_b, gsem_b).wait()

            @pl.when(j < CH // 2 - 1)
            def _():
                pltpu.async_copy(y_hbm.at[sidx_v.at[p, 2 * j + 2]], rows_a,
                                 gsem_a)
            pltpu.sync_copy(rows_b, acc_s.at[didx_v.at[p, rb]], add=True)
            return 0
        lax.fori_loop(0, CH // 2, _pair, 0)

        if nxt < NCH:
            q = nxt % 2
            pltpu.make_async_copy(srcr.at[wid, pl.ds(nxt * CH, CH)],
                                  sidx_v.at[q], isem_s).wait()
            pltpu.make_async_copy(dstr.at[wid, pl.ds(nxt * CH, CH)],
                                  didx_v.at[q], isem_d).wait()
            pltpu.async_copy(y_hbm.at[sidx_v.at[q, 0]], rows_a, gsem_a)

    plsc.subcore_barrier()
    for t in range(RPT // C):
        sl = pl.ds(base + t * C, C)
        pltpu.sync_copy(acc_s.at[sl], accp.at[c, sl])


def _tc1_body(x_ref, w_ref, degp_ref, xw_ref, y_ref):
    deg = jnp.sum(degp_ref[...], axis=0) + 1.0
    dinv = lax.rsqrt(deg)[:, None]
    xw = jnp.dot(x_ref[...], w_ref[...], preferred_element_type=jnp.float32)
    xw_ref[...] = xw
    y_ref[...] = xw * dinv


def _tc2_body(accp_ref, xw_ref, degp_ref, b_ref, w_ref, xw2_ref, y2_ref):
    deg = jnp.sum(degp_ref[...], axis=0) + 1.0
    dinv = lax.rsqrt(deg)[:, None]
    acc = jnp.sum(accp_ref[...], axis=0)
    h = jnp.maximum(acc * dinv + xw_ref[...] * (dinv * dinv) + b_ref[...], 0.0)
    xw2 = jnp.dot(h, w_ref[...], preferred_element_type=jnp.float32)
    xw2_ref[...] = xw2
    y2_ref[...] = xw2 * dinv


def _tc3_body(accp_ref, xw_ref, degp_ref, b_ref, seg_ref, wout_ref, bout_ref,
              out_ref, sums_s, cnt_s):
    i = pl.program_id(0)

    @pl.when(i == 0)
    def _():
        sums_s[...] = jnp.zeros_like(sums_s)
        cnt_s[...] = jnp.zeros_like(cnt_s)

    deg = jnp.sum(degp_ref[...], axis=0) + 1.0
    dinv = lax.rsqrt(deg)[:, None]
    acc = jnp.sum(accp_ref[...], axis=0)
    h = jnp.maximum(acc * dinv + xw_ref[...] * (dinv * dinv) + b_ref[...], 0.0)
    seg = seg_ref[...]
    segid = lax.broadcasted_iota(jnp.int32, (M, BLK), 0)
    onehot_t = jnp.where(segid == seg, 1.0, 0.0)
    sums_s[...] += jnp.dot(onehot_t, h, preferred_element_type=jnp.float32)
    cnt_s[...] += jnp.broadcast_to(
        jnp.sum(onehot_t, axis=1, keepdims=True), (M, D))

    @pl.when(i == pl.num_programs(0) - 1)
    def _():
        z = sums_s[...] / jnp.maximum(cnt_s[...], 1.0)
        out_ref[...] = (
            jnp.dot(z, wout_ref[...], preferred_element_type=jnp.float32)
            + bout_ref[...])


_row_spec = pl.BlockSpec((BLK, D), lambda i: (i, 0))
_w_spec = pl.BlockSpec((D, D), lambda i: (0, 0))
_deg_spec = pl.BlockSpec((NC, BLK), lambda i: (0, i))
_acc_spec = pl.BlockSpec((NC, BLK, D), lambda i: (0, i, 0))
_b_spec = pl.BlockSpec((1, D), lambda i: (0, 0))

_tc1 = pl.pallas_call(
    _tc1_body,
    grid=(GRID,),
    in_specs=[_row_spec, _w_spec, _deg_spec],
    out_specs=[_row_spec, _row_spec],
    out_shape=[jax.ShapeDtypeStruct((NP, D), jnp.float32)] * 2,
)

_tc2 = pl.pallas_call(
    _tc2_body,
    grid=(GRID,),
    in_specs=[_acc_spec, _row_spec, _deg_spec, _b_spec, _w_spec],
    out_specs=[_row_spec, _row_spec],
    out_shape=[jax.ShapeDtypeStruct((NP, D), jnp.float32)] * 2,
)

_tc3 = pl.pallas_call(
    _tc3_body,
    grid=(GRID,),
    in_specs=[_acc_spec, _row_spec, _deg_spec, _b_spec,
              pl.BlockSpec((1, BLK), lambda i: (0, i)), _w_spec, _b_spec],
    out_specs=pl.BlockSpec((M, D), lambda i: (0, 0)),
    out_shape=jax.ShapeDtypeStruct((M, D), jnp.float32),
    scratch_shapes=[pltpu.VMEM((M, D), jnp.float32),
                    pltpu.VMEM((M, D), jnp.float32)],
)


def kernel(x, edge_index, batch_idx, W1, b1, W2, b2, Wout, bout):
    pad = N + (jnp.arange(EP - E, dtype=jnp.int32) % (NP - N))
    src = jnp.concatenate([edge_index[0], pad])
    dst = jnp.concatenate([edge_index[1], pad])
    srcr = src.reshape(NW, NB, C)
    dstr = dst.reshape(NW, NB, C)
    dsth = dst.reshape(NW, RD, C)
    x_pad = jnp.pad(x, ((0, NP - N), (0, 0)))
    seg_pad = jnp.pad(batch_idx, (0, NP - N), constant_values=M).reshape(1, NP)
    b1r = b1.reshape(1, D)
    b2r = b2.reshape(1, D)
    boutr = bout.reshape(1, D)

    degp = _deg_kernel(dsth).reshape(NC, NP)
    xw1, y1 = _tc1(x_pad, W1, degp)
    acc1 = _prop_kernel(y1, srcr, dstr)
    xw2, y2 = _tc2(acc1, xw1, degp, b1r, W2)
    acc2 = _prop_kernel(y2, srcr, dstr)
    return _tc3(acc2, xw2, degp, b2r, seg_pad, Wout, boutr)
